# Initial kernel scaffold; baseline (speedup 1.0000x reference)
import jax
import jax.numpy as jnp
from jax.experimental import pallas as pl


def kernel(token_ids, embeddings):
    raise NotImplementedError("write your pallas kernel here")


# speedup vs baseline: 3.5430x; 3.5430x over previous
import functools

import jax
import jax.numpy as jnp
from jax import lax
from jax.experimental import pallas as pl
from jax.experimental.pallas import tpu as pltpu
from jax.experimental.pallas import tpu_sc as plsc

NC = 2
NS = 16
NW = NC * NS
CHUNK = 128


def _emb_call(n_chunks, D):
    mesh = plsc.VectorSubcoreMesh(core_axis_name="c", subcore_axis_name="s")

    @functools.partial(
        pl.kernel,
        out_type=jax.ShapeDtypeStruct((NW, n_chunks, CHUNK, D), jnp.float32),
        mesh=mesh,
        scratch_types=[
            pltpu.VMEM((n_chunks, CHUNK), jnp.int32),
            pltpu.VMEM((CHUNK, D), jnp.float32),
            pltpu.SemaphoreType.DMA,
        ],
        compiler_params=pltpu.CompilerParams(use_tc_tiling_on_sc=False),
    )
    def emb_kernel(idx_hbm, table_hbm, out_hbm, idx_v, rows_v, gsem):
        wid = lax.axis_index("s") * NC + lax.axis_index("c")
        pltpu.sync_copy(idx_hbm.at[wid], idx_v)

        def body(j, carry):
            pltpu.async_copy(table_hbm.at[idx_v.at[j]], rows_v, gsem).wait()
            pltpu.sync_copy(rows_v, out_hbm.at[wid, j])
            return carry

        lax.fori_loop(0, n_chunks, body, 0)

    return emb_kernel


def kernel(token_ids, embeddings):
    B, S = token_ids.shape
    V, D = embeddings.shape
    total = B * S
    assert total % (NW * CHUNK) == 0
    n_chunks = total // (NW * CHUNK)

    idx = token_ids.reshape(NW, n_chunks, CHUNK).astype(jnp.int32)
    out = _emb_call(n_chunks, D)(idx, embeddings)
    return out.reshape(B, S, D)


# speedup vs baseline: 4.2466x; 1.1986x over previous
import functools

import jax
import jax.numpy as jnp
from jax import lax
from jax.experimental import pallas as pl
from jax.experimental.pallas import tpu as pltpu
from jax.experimental.pallas import tpu_sc as plsc

NC = 2
NS = 16
NW = NC * NS
CHUNK = 128


NBUF = 8


def _emb_call(n_chunks, D):
    mesh = plsc.VectorSubcoreMesh(core_axis_name="c", subcore_axis_name="s")

    @functools.partial(
        pl.kernel,
        out_type=jax.ShapeDtypeStruct((NW, n_chunks, CHUNK, D), jnp.float32),
        mesh=mesh,
        scratch_types=[
            pltpu.VMEM((n_chunks, CHUNK), jnp.int32),
            pltpu.VMEM((NBUF, CHUNK, D), jnp.float32),
            pltpu.SemaphoreType.DMA((NBUF,)),
            pltpu.SemaphoreType.DMA((NBUF,)),
        ],
        compiler_params=pltpu.CompilerParams(use_tc_tiling_on_sc=False),
    )
    def emb_kernel(idx_hbm, table_hbm, out_hbm, idx_v, rows_v, gsem, osem):
        wid = lax.axis_index("s") * NC + lax.axis_index("c")
        pltpu.sync_copy(idx_hbm.at[wid], idx_v)

        def gfire(b, j):
            pltpu.async_copy(table_hbm.at[idx_v.at[j]], rows_v.at[b], gsem.at[b])

        def gwait(b, j):
            pltpu.make_async_copy(
                table_hbm.at[idx_v.at[j]], rows_v.at[b], gsem.at[b]
            ).wait()

        def ofire(b, j):
            pltpu.async_copy(rows_v.at[b], out_hbm.at[wid, j], osem.at[b])

        def owait(b, j):
            pltpu.make_async_copy(
                rows_v.at[b], out_hbm.at[wid, j], osem.at[b]
            ).wait()

        for b in range(NBUF):
            gfire(b, b)

        def outer(h, carry):
            j0 = h * NBUF
            for b in range(NBUF):
                gwait(b, j0 + b)
                ofire(b, j0 + b)
            for b in range(NBUF):
                owait(b, j0 + b)
                jn = j0 + b + NBUF

                @pl.when(jn < n_chunks)
                def _():
                    gfire(b, jn)

            return carry

        lax.fori_loop(0, n_chunks // NBUF, outer, 0)

    return emb_kernel


def kernel(token_ids, embeddings):
    B, S = token_ids.shape
    V, D = embeddings.shape
    total = B * S
    assert total % (NW * CHUNK * NBUF) == 0
    n_chunks = total // (NW * CHUNK)

    idx = token_ids.reshape(NW, n_chunks, CHUNK).astype(jnp.int32)
    out = _emb_call(n_chunks, D)(idx, embeddings)
    return out.reshape(B, S, D)
